# Initial kernel scaffold; baseline (speedup 1.0000x reference)
import jax
import jax.numpy as jnp
from jax.experimental import pallas as pl


def kernel(x, edge_index, W0, b0, W1, b1, W2, b2, W3, b3, g0, be0, g1, be1, g2, be2, Lw0, Lb0, Lw1, Lb1, Lw2, Lb2):
    raise NotImplementedError("write your pallas kernel here")


# speedup vs baseline: 4.6074x; 4.6074x over previous
import functools

import jax
import jax.numpy as jnp
from jax import lax
from jax.experimental import pallas as pl
from jax.experimental.pallas import tpu as pltpu
from jax.experimental.pallas import tpu_sc as plsc

N = 10000
NP = 10240
E = 160000
DIN = 256
DH = 512
DOUT = 256
EPS = 1e-5
NC = 2
NS = 16
EPW = E // (NC * NS)
NW = NC * NS
HALF = NP // 2
CAP = 5248
ACCR = HALF + 128
RB = 512
GR = NP // RB
NCH = 4

@functools.cache
def _mesh():
    return plsc.VectorSubcoreMesh(core_axis_name="c", subcore_axis_name="s",
                                  num_cores=NC, num_subcores=NS)


def _zero_hist(hist_v, nwords):
    def zero(i, carry):
        hist_v[pl.ds(i * 16, 16)] = jnp.zeros((16,), jnp.float32)
        return carry

    lax.fori_loop(0, nwords // 16, zero, 0)


def _deg_body(dst_hbm, out_hbm, idx_v, hist_v):
    c = lax.axis_index("c")
    s = lax.axis_index("s")
    pltpu.sync_copy(dst_hbm.at[c, s], idx_v)
    _zero_hist(hist_v, NP + 16)
    one0 = jnp.where(lax.iota(jnp.int32, 16) == 0, 1.0, 0.0)

    def body(i, carry):
        dv = idx_v[pl.ds(i * 16, 16)]
        for k in range(16):
            d = dv[k]
            hist_v[pl.ds(d, 16)] = hist_v[pl.ds(d, 16)] + one0
        return carry

    lax.fori_loop(0, EPW // 16, body, 0)
    pltpu.sync_copy(hist_v.at[pl.ds(0, NP)], out_hbm.at[c, s])


@functools.cache
def _deg_call():
    return pl.kernel(
        _deg_body,
        out_type=jax.ShapeDtypeStruct((NC, NS, NP), jnp.float32),
        mesh=_mesh(),
        scratch_types=[
            pltpu.VMEM((EPW,), jnp.int32),
            pltpu.VMEM((NP + 16,), jnp.float32),
        ],
    )


def _s_body(src_hbm, dst_hbm, dinv_hbm, out_hbm, sidx_v, didx_v, dinv_v, hist_v):
    c = lax.axis_index("c")
    s = lax.axis_index("s")
    pltpu.sync_copy(src_hbm.at[c, s], sidx_v)
    pltpu.sync_copy(dst_hbm.at[c, s], didx_v)
    pltpu.sync_copy(dinv_hbm, dinv_v.at[pl.ds(0, NP)])
    _zero_hist(hist_v, NP + 16)
    io16 = lax.iota(jnp.int32, 16)

    def body(i, carry):
        sv = sidx_v[pl.ds(i * 16, 16)]
        dv = didx_v[pl.ds(i * 16, 16)]
        for k in range(16):
            s0 = sv[k]
            dval = dinv_v[pl.ds(dv[k], 16)][0]
            upd = jnp.where(io16 == 0, dval, 0.0)
            hist_v[pl.ds(s0, 16)] = hist_v[pl.ds(s0, 16)] + upd
        return carry

    lax.fori_loop(0, EPW // 16, body, 0)
    pltpu.sync_copy(hist_v.at[pl.ds(0, NP)], out_hbm.at[c, s])


@functools.cache
def _s_call():
    return pl.kernel(
        _s_body,
        out_type=jax.ShapeDtypeStruct((NC, NS, NP), jnp.float32),
        mesh=_mesh(),
        scratch_types=[
            pltpu.VMEM((EPW,), jnp.int32),
            pltpu.VMEM((EPW,), jnp.int32),
            pltpu.VMEM((NP + 16,), jnp.float32),
            pltpu.VMEM((NP + 16,), jnp.float32),
        ],
    )


def _part_body(idx_hbm, psrc_hbm, pdst_hbm, cnt_hbm,
               sidx_v, didx_v, lsrc_v, ldst_v, hsrc_v, hdst_v, cnt_v):
    c = lax.axis_index("c")
    s = lax.axis_index("s")
    w = c * NS + s
    pltpu.sync_copy(idx_hbm.at[0, c, s], sidx_v)
    pltpu.sync_copy(idx_hbm.at[1, c, s], didx_v)
    io16 = lax.iota(jnp.int32, 16)

    def body(i, offs):
        olo, ohi = offs
        sv = sidx_v[pl.ds(i * 16, 16)]
        dv = didx_v[pl.ds(i * 16, 16)]
        for k in range(16):
            sk = sv[k]
            dk = dv[k]
            is_lo = (dk < HALF).astype(jnp.int32)
            dl = jnp.where(dk < HALF, dk, dk - HALF)
            sspl = jnp.where(io16 == 0, sk, 0)
            dspl = jnp.where(io16 == 0, dl, 0)
            lsrc_v[pl.ds(olo, 16)] = sspl
            ldst_v[pl.ds(olo, 16)] = dspl
            hsrc_v[pl.ds(ohi, 16)] = sspl
            hdst_v[pl.ds(ohi, 16)] = dspl
            olo = olo + is_lo
            ohi = ohi + (1 - is_lo)
        return olo, ohi

    olo, ohi = lax.fori_loop(0, EPW // 16, body,
                             (jnp.int32(0), jnp.int32(0)))
    zsrc = jnp.zeros((16,), jnp.int32)
    for t in range(8):
        trash = HALF + io16 + 16 * t
        lsrc_v[pl.ds(olo + t * 16, 16)] = zsrc
        ldst_v[pl.ds(olo + t * 16, 16)] = trash
        hsrc_v[pl.ds(ohi + t * 16, 16)] = zsrc
        hdst_v[pl.ds(ohi + t * 16, 16)] = trash
    def czero(i, carry):
        cnt_v[pl.ds(i * 16, 16)] = jnp.zeros((16,), jnp.int32)
        return carry

    lax.fori_loop(0, 16, czero, 0)
    cnt_v[pl.ds(0, 16)] = jnp.where(io16 == 0, olo, 0)
    cnt_v[pl.ds(128, 16)] = jnp.where(io16 == 0, ohi, 0)
    pltpu.sync_copy(lsrc_v, psrc_hbm.at[0, w])
    pltpu.sync_copy(ldst_v, pdst_hbm.at[0, w])
    pltpu.sync_copy(hsrc_v, psrc_hbm.at[1, w])
    pltpu.sync_copy(hdst_v, pdst_hbm.at[1, w])
    pltpu.sync_copy(cnt_v.at[pl.ds(0, 128)], cnt_hbm.at[0, w])
    pltpu.sync_copy(cnt_v.at[pl.ds(128, 128)], cnt_hbm.at[1, w])


@functools.cache
def _part_call():
    return pl.kernel(
        _part_body,
        out_type=(
            jax.ShapeDtypeStruct((2, NW, CAP), jnp.int32),
            jax.ShapeDtypeStruct((2, NW, CAP), jnp.int32),
            jax.ShapeDtypeStruct((2, NW, 128), jnp.int32),
        ),
        mesh=_mesh(),
        scratch_types=[
            pltpu.VMEM((EPW,), jnp.int32),
            pltpu.VMEM((EPW,), jnp.int32),
            pltpu.VMEM((CAP,), jnp.int32),
            pltpu.VMEM((CAP,), jnp.int32),
            pltpu.VMEM((CAP,), jnp.int32),
            pltpu.VMEM((CAP,), jnp.int32),
            pltpu.VMEM((256,), jnp.int32),
        ],
    )


def _agg_body(pp_hbm, psrc_hbm, pdst_hbm, cnt_hbm, out_hbm,
              sidx_v, didx_v, cnt_v, rows_v, zbuf_v, acc_sh):
    c = lax.axis_index("c")
    s = lax.axis_index("s")

    def zfill(r, carry):
        for k in range(8):
            zbuf_v[r, pl.ds(k * 16, 16)] = jnp.zeros((16,), jnp.float32)
        return carry

    lax.fori_loop(0, 128, zfill, 0)

    def run_half(bkt, cbase):
        for j in range(2):
            w = 2 * s + j
            pltpu.sync_copy(psrc_hbm.at[bkt, w], sidx_v.at[j])
            pltpu.sync_copy(pdst_hbm.at[bkt, w], didx_v.at[j])
            pltpu.sync_copy(cnt_hbm.at[bkt, w], cnt_v.at[pl.ds(128 * j, 128)])

        for f in range(NCH):
            pp = pp_hbm.at[f]
            out = out_hbm.at[f]
            pltpu.sync_copy(zbuf_v, acc_sh.at[pl.ds(s * 328, 128)])
            pltpu.sync_copy(zbuf_v, acc_sh.at[pl.ds(s * 328 + 128, 128)])
            pltpu.sync_copy(zbuf_v.at[pl.ds(0, 72)],
                            acc_sh.at[pl.ds(s * 328 + 256, 72)])
            plsc.subcore_barrier()
            for j in range(2):
                n = cnt_v[pl.ds(128 * j, 16)][0]
                nb = (n + 127) // 128

                def blk(b, carry):
                    pltpu.sync_copy(pp.at[sidx_v.at[j, b]], rows_v)
                    pltpu.sync_copy(rows_v, acc_sh.at[didx_v.at[j, b]],
                                    add=True)
                    return carry

                lax.fori_loop(0, nb, blk, 0)
            plsc.subcore_barrier()
            sl = pl.ds(s * 320, 320)
            pltpu.sync_copy(acc_sh.at[sl], out.at[pl.ds(cbase + s * 320, 320)])
            plsc.subcore_barrier()

    @pl.when(c == 0)
    def _():
        run_half(0, 0)

    @pl.when(c == 1)
    def _():
        run_half(1, HALF)


@functools.cache
def _agg_call():
    return pl.kernel(
        _agg_body,
        out_type=jax.ShapeDtypeStruct((NCH, NP, 128), jnp.float32),
        mesh=_mesh(),
        scratch_types=[
            pltpu.VMEM((2, CAP // 128, 128), jnp.int32),
            pltpu.VMEM((2, CAP // 128, 128), jnp.int32),
            pltpu.VMEM((256,), jnp.int32),
            pltpu.VMEM((128, 128), jnp.float32),
            pltpu.VMEM((128, 128), jnp.float32),
            pltpu.VMEM_SHARED((ACCR, 128), jnp.float32),
        ],
    )


def _dinv_body(dp_ref, out_ref):
    out_ref[...] = lax.rsqrt(jnp.sum(dp_ref[...], axis=0) + 1.0)


def _dinv_call(dp):
    return pl.pallas_call(
        _dinv_body,
        out_shape=jax.ShapeDtypeStruct((NP // 128, 128), jnp.float32),
    )(dp)


def _w_body(sp_ref, dinv_ref, out_ref):
    ssum = jnp.sum(sp_ref[...], axis=0)
    dinv = dinv_ref[...]
    r = lax.broadcasted_iota(jnp.int32, (NP // 128, 128), 0)
    l = lax.broadcasted_iota(jnp.int32, (NP // 128, 128), 1)
    valid = (r * 128 + l) < N
    w = dinv * (ssum + dinv) * (1.0 / N)
    out_ref[...] = jnp.where(valid, w, 0.0)


def _w_call(sp, dinv):
    return pl.pallas_call(
        _w_body,
        out_shape=jax.ShapeDtypeStruct((NP // 128, 128), jnp.float32),
    )(sp, dinv)


def _mm_body(h_ref, wcat_ref, dvr_ref, lb_ref, pp_ref, r_ref):
    prod = jnp.dot(h_ref[...], wcat_ref[...], preferred_element_type=jnp.float32)
    dvr = dvr_ref[...]
    for f in range(NCH):
        pp_ref[f] = prod[:, 128 * f:128 * (f + 1)] * dvr
    r_ref[...] = prod[:, DH:] + lb_ref[...].reshape(1, DH)


def _mm_call(h, wcat, dvr, lb):
    k = h.shape[1]
    return pl.pallas_call(
        _mm_body,
        grid=(GR,),
        in_specs=[
            pl.BlockSpec((RB, k), lambda i: (i, 0)),
            pl.BlockSpec((k, DH * 2), lambda i: (0, 0)),
            pl.BlockSpec((RB, 128), lambda i: (i, 0)),
            pl.BlockSpec((1, 1, DH), lambda i: (0, 0, 0)),
        ],
        out_specs=(
            pl.BlockSpec((NCH, RB, 128), lambda i: (0, i, 0)),
            pl.BlockSpec((RB, DH), lambda i: (i, 0)),
        ),
        out_shape=(
            jax.ShapeDtypeStruct((NCH, NP, 128), jnp.float32),
            jax.ShapeDtypeStruct((NP, DH), jnp.float32),
        ),
        compiler_params=pltpu.CompilerParams(dimension_semantics=("arbitrary",)),
    )(h, wcat, dvr, lb)


def _bn_body(sc_ref, pp_ref, dvr_ref, r_ref, g_ref, be_ref, wrep_ref,
             h_ref, wsum_ref, stats, *, last):
    p = pl.program_id(0)
    i = pl.program_id(1)
    dvr = dvr_ref[...]
    y = jnp.concatenate(
        [(sc_ref[f] + pp_ref[f]) * dvr for f in range(NCH)], axis=1)

    @pl.when(jnp.logical_and(p == 0, i == 0))
    def _():
        stats[...] = jnp.zeros_like(stats)

    @pl.when(p == 0)
    def _():
        rows = i * RB + lax.broadcasted_iota(jnp.int32, (RB, DH), 0)
        ym = jnp.where(rows < N, y, 0.0)
        stats[0:1, :] = stats[0:1, :] + jnp.sum(ym, axis=0, keepdims=True)
        stats[1:2, :] = stats[1:2, :] + jnp.sum(ym * ym, axis=0, keepdims=True)

    @pl.when(p == 1)
    def _():
        m = stats[0:1, :] * (1.0 / N)
        v = stats[1:2, :] * (1.0 / N) - m * m
        rstd = lax.rsqrt(v + EPS)
        g = g_ref[...].reshape(1, DH)
        be = be_ref[...].reshape(1, DH)
        hn = jnp.maximum((y - m) * rstd * g + be, 0.0) + r_ref[...]
        h_ref[...] = hn
        if last:
            wr = wrep_ref[...]
            wcat = jnp.concatenate([wr, wr, wr, wr], axis=1)
            stats[2:3, :] = stats[2:3, :] + jnp.sum(hn * wcat, axis=0, keepdims=True)

            @pl.when(i == GR - 1)
            def _():
                wsum_ref[...] = stats[...]


def _bn_call(scs, pps, dvr, r, g, be, wrep, *, last):
    def body(sc_ref, pp_ref, dv, rr, gg, bb, *rest):
        if last:
            (wrp, h_ref, wsum_ref, stats) = rest
        else:
            (h_ref, stats) = rest
            wrp, wsum_ref = None, None
        _bn_body(sc_ref, pp_ref, dv, rr, gg, bb, wrp,
                 h_ref, wsum_ref, stats, last=last)

    chunk_spec = pl.BlockSpec((RB, 128), lambda p, i: (i, 0))
    stack_spec = pl.BlockSpec((NCH, RB, 128), lambda p, i: (0, i, 0))
    in_specs = (
        [stack_spec, stack_spec]
        + [chunk_spec,
           pl.BlockSpec((RB, DH), lambda p, i: (i, 0)),
           pl.BlockSpec((1, 1, DH), lambda p, i: (0, 0, 0)),
           pl.BlockSpec((1, 1, DH), lambda p, i: (0, 0, 0))]
    )
    out_specs = [pl.BlockSpec((RB, DH), lambda p, i: (p * i, 0))]
    out_shape = [jax.ShapeDtypeStruct((NP, DH), jnp.float32)]
    args = [scs, pps, dvr, r, g, be]
    if last:
        in_specs.append(chunk_spec)
        args.append(wrep)
        out_specs.append(pl.BlockSpec((8, DH), lambda p, i: (0, 0)))
        out_shape.append(jax.ShapeDtypeStruct((8, DH), jnp.float32))
    res = pl.pallas_call(
        body,
        grid=(2, GR),
        in_specs=in_specs,
        out_specs=tuple(out_specs),
        out_shape=tuple(out_shape),
        scratch_shapes=[pltpu.VMEM((8, DH), jnp.float32)],
        compiler_params=pltpu.CompilerParams(
            dimension_semantics=("arbitrary", "arbitrary")),
    )(*args)
    return res if last else (res[0], None)


def _fin_body(ws_ref, w3_ref, b3_ref, out_ref):
    v = ws_ref[2:3, :]
    res = jnp.dot(v, w3_ref[...], preferred_element_type=jnp.float32)
    res = res + b3_ref[...].reshape(1, DOUT)
    out_ref[...] = jnp.broadcast_to(res, (8, DOUT))


def _fin_call(wsum, W3, b3):
    return pl.pallas_call(
        _fin_body,
        out_shape=jax.ShapeDtypeStruct((8, DOUT), jnp.float32),
    )(wsum, W3, b3)


def kernel(x, edge_index, W0, b0, W1, b1, W2, b2, W3, b3,
           g0, be0, g1, be1, g2, be2, Lw0, Lb0, Lw1, Lb1, Lw2, Lb2):
    src = edge_index[0]
    dst = edge_index[1]
    src_w = src.reshape(NC, NS, EPW)
    dst_w = dst.reshape(NC, NS, EPW)
    idx_w = edge_index.reshape(2, NC, NS, EPW)
    psrc, pdst, cnts = _part_call()(idx_w)
    psrc = psrc.reshape(2, NW, CAP // 128, 128)
    pdst = pdst.reshape(2, NW, CAP // 128, 128)

    deg_parts = _deg_call()(dst_w)
    dinv2d = _dinv_call(deg_parts.reshape(NC * NS, NP // 128, 128))
    dinv_lin = dinv2d.reshape(NP)
    dvr = jnp.broadcast_to(dinv_lin[:, None], (NP, 128))
    s_parts = _s_call()(src_w, dst_w, dinv_lin)
    w2d = _w_call(s_parts.reshape(NC * NS, NP // 128, 128), dinv2d)
    wrep = jnp.broadcast_to(w2d.reshape(NP)[:, None], (NP, 128))

    h = jnp.pad(x, ((0, NP - N), (0, 0)))
    layer_params = ((W0, Lw0, Lb0, g0, be0),
                    (W1, Lw1, Lb1, g1, be1),
                    (W2, Lw2, Lb2, g2, be2))
    wsum = None
    for li, (W, Lw, Lb, g, be) in enumerate(layer_params):
        wcat = jnp.concatenate([W, Lw], axis=1)
        pps, r = _mm_call(h, wcat, dvr, Lb.reshape(1, 1, DH))
        scs = _agg_call()(pps, psrc, pdst, cnts)
        h, wsum = _bn_call(scs, pps, dvr, r, g.reshape(1, 1, DH),
                           be.reshape(1, 1, DH), wrep, last=(li == 2))

    out = _fin_call(wsum, W3, b3.reshape(1, 1, DOUT))
    return out[0:1]


# speedup vs baseline: 4.8775x; 1.0586x over previous
import functools

import jax
import jax.numpy as jnp
from jax import lax
from jax.experimental import pallas as pl
from jax.experimental.pallas import tpu as pltpu
from jax.experimental.pallas import tpu_sc as plsc

N = 10000
NP = 10240
E = 160000
DIN = 256
DH = 512
DOUT = 256
EPS = 1e-5
NC = 2
NS = 16
EPW = E // (NC * NS)
NW = NC * NS
HALF = NP // 2
CAP = 5248
ACCR = HALF + 128
RB = 512
GR = NP // RB
NCH = 4

@functools.cache
def _mesh():
    return plsc.VectorSubcoreMesh(core_axis_name="c", subcore_axis_name="s",
                                  num_cores=NC, num_subcores=NS)


def _zero_hist(hist_v, nwords):
    def zero(i, carry):
        hist_v[pl.ds(i * 16, 16)] = jnp.zeros((16,), jnp.float32)
        return carry

    lax.fori_loop(0, nwords // 16, zero, 0)


def _deg_body(dst_hbm, out_hbm, idx_v, hist_v):
    c = lax.axis_index("c")
    s = lax.axis_index("s")
    pltpu.sync_copy(dst_hbm.at[c, s], idx_v)
    _zero_hist(hist_v, NP + 16)
    one0 = jnp.where(lax.iota(jnp.int32, 16) == 0, 1.0, 0.0)

    def body(i, carry):
        dv = idx_v[pl.ds(i * 16, 16)]
        for k in range(16):
            d = dv[k]
            hist_v[pl.ds(d, 16)] = hist_v[pl.ds(d, 16)] + one0
        return carry

    lax.fori_loop(0, EPW // 16, body, 0)
    pltpu.sync_copy(hist_v.at[pl.ds(0, NP)], out_hbm.at[c, s])


@functools.cache
def _deg_call():
    return pl.kernel(
        _deg_body,
        out_type=jax.ShapeDtypeStruct((NC, NS, NP), jnp.float32),
        mesh=_mesh(),
        scratch_types=[
            pltpu.VMEM((EPW,), jnp.int32),
            pltpu.VMEM((NP + 16,), jnp.float32),
        ],
    )


def _s_body(src_hbm, dst_hbm, dinv_hbm, out_hbm, sidx_v, didx_v, dinv_v, hist_v):
    c = lax.axis_index("c")
    s = lax.axis_index("s")
    pltpu.sync_copy(src_hbm.at[c, s], sidx_v)
    pltpu.sync_copy(dst_hbm.at[c, s], didx_v)
    pltpu.sync_copy(dinv_hbm, dinv_v.at[pl.ds(0, NP)])
    _zero_hist(hist_v, NP + 16)
    io16 = lax.iota(jnp.int32, 16)

    def body(i, carry):
        sv = sidx_v[pl.ds(i * 16, 16)]
        dv = didx_v[pl.ds(i * 16, 16)]
        for k in range(16):
            s0 = sv[k]
            dval = dinv_v[pl.ds(dv[k], 16)][0]
            upd = jnp.where(io16 == 0, dval, 0.0)
            hist_v[pl.ds(s0, 16)] = hist_v[pl.ds(s0, 16)] + upd
        return carry

    lax.fori_loop(0, EPW // 16, body, 0)
    pltpu.sync_copy(hist_v.at[pl.ds(0, NP)], out_hbm.at[c, s])


@functools.cache
def _s_call():
    return pl.kernel(
        _s_body,
        out_type=jax.ShapeDtypeStruct((NC, NS, NP), jnp.float32),
        mesh=_mesh(),
        scratch_types=[
            pltpu.VMEM((EPW,), jnp.int32),
            pltpu.VMEM((EPW,), jnp.int32),
            pltpu.VMEM((NP + 16,), jnp.float32),
            pltpu.VMEM((NP + 16,), jnp.float32),
        ],
    )


def _part_body(idx_hbm, psrc_hbm, pdst_hbm, cnt_hbm,
               sidx_v, didx_v, lsrc_v, ldst_v, hsrc_v, hdst_v, cnt_v):
    c = lax.axis_index("c")
    s = lax.axis_index("s")
    w = c * NS + s
    pltpu.sync_copy(idx_hbm.at[0, c, s], sidx_v)
    pltpu.sync_copy(idx_hbm.at[1, c, s], didx_v)
    io16 = lax.iota(jnp.int32, 16)

    def body(i, offs):
        olo, ohi = offs
        sv = sidx_v[pl.ds(i * 16, 16)]
        dv = didx_v[pl.ds(i * 16, 16)]
        for k in range(16):
            sk = sv[k]
            dk = dv[k]
            is_lo = (dk < HALF).astype(jnp.int32)
            dl = jnp.where(dk < HALF, dk, dk - HALF)
            sspl = jnp.where(io16 == 0, sk, 0)
            dspl = jnp.where(io16 == 0, dl, 0)
            lsrc_v[pl.ds(olo, 16)] = sspl
            ldst_v[pl.ds(olo, 16)] = dspl
            hsrc_v[pl.ds(ohi, 16)] = sspl
            hdst_v[pl.ds(ohi, 16)] = dspl
            olo = olo + is_lo
            ohi = ohi + (1 - is_lo)
        return olo, ohi

    olo, ohi = lax.fori_loop(0, EPW // 16, body,
                             (jnp.int32(0), jnp.int32(0)))
    zsrc = jnp.zeros((16,), jnp.int32)
    for t in range(8):
        trash = HALF + io16 + 16 * t
        lsrc_v[pl.ds(olo + t * 16, 16)] = zsrc
        ldst_v[pl.ds(olo + t * 16, 16)] = trash
        hsrc_v[pl.ds(ohi + t * 16, 16)] = zsrc
        hdst_v[pl.ds(ohi + t * 16, 16)] = trash
    def czero(i, carry):
        cnt_v[pl.ds(i * 16, 16)] = jnp.zeros((16,), jnp.int32)
        return carry

    lax.fori_loop(0, 16, czero, 0)
    cnt_v[pl.ds(0, 16)] = jnp.where(io16 == 0, olo, 0)
    cnt_v[pl.ds(128, 16)] = jnp.where(io16 == 0, ohi, 0)
    pltpu.sync_copy(lsrc_v, psrc_hbm.at[0, w])
    pltpu.sync_copy(ldst_v, pdst_hbm.at[0, w])
    pltpu.sync_copy(hsrc_v, psrc_hbm.at[1, w])
    pltpu.sync_copy(hdst_v, pdst_hbm.at[1, w])
    pltpu.sync_copy(cnt_v.at[pl.ds(0, 128)], cnt_hbm.at[0, w])
    pltpu.sync_copy(cnt_v.at[pl.ds(128, 128)], cnt_hbm.at[1, w])


@functools.cache
def _part_call():
    return pl.kernel(
        _part_body,
        out_type=(
            jax.ShapeDtypeStruct((2, NW, CAP), jnp.int32),
            jax.ShapeDtypeStruct((2, NW, CAP), jnp.int32),
            jax.ShapeDtypeStruct((2, NW, 128), jnp.int32),
        ),
        mesh=_mesh(),
        scratch_types=[
            pltpu.VMEM((EPW,), jnp.int32),
            pltpu.VMEM((EPW,), jnp.int32),
            pltpu.VMEM((CAP,), jnp.int32),
            pltpu.VMEM((CAP,), jnp.int32),
            pltpu.VMEM((CAP,), jnp.int32),
            pltpu.VMEM((CAP,), jnp.int32),
            pltpu.VMEM((256,), jnp.int32),
        ],
    )


NSLOT = 3


def _agg_body(pp_hbm, psrc_hbm, pdst_hbm, cnt_hbm, out_hbm,
              sidx_v, didx_v, cnt_v, r0, r1, r2, zbuf_v, acc_sh,
              g0, g1, g2, s0, s1, s2):
    c = lax.axis_index("c")
    s = lax.axis_index("s")
    rows = (r0, r1, r2)
    gsem = (g0, g1, g2)
    ssem = (s0, s1, s2)

    def zfill(r, carry):
        for k in range(8):
            zbuf_v[r, pl.ds(k * 16, 16)] = jnp.zeros((16,), jnp.float32)
        return carry

    lax.fori_loop(0, 64, zfill, 0)

    def run_half(bkt, cbase):
        for j in range(2):
            w = 2 * s + j
            pltpu.sync_copy(psrc_hbm.at[bkt, w], sidx_v.at[j])
            pltpu.sync_copy(pdst_hbm.at[bkt, w], didx_v.at[j])
            pltpu.sync_copy(cnt_hbm.at[bkt, w], cnt_v.at[pl.ds(128 * j, 128)])

        for f in range(NCH):
            pp = pp_hbm.at[f]
            out = out_hbm.at[f]
            dummy = pp.at[pl.ds(0, 128)]
            for z in range(5):
                pltpu.sync_copy(zbuf_v, acc_sh.at[pl.ds(s * 328 + z * 64, 64)])
            pltpu.sync_copy(zbuf_v.at[pl.ds(0, 8)],
                            acc_sh.at[pl.ds(s * 328 + 320, 8)])
            plsc.subcore_barrier()
            for j in range(2):
                n = cnt_v[pl.ds(128 * j, 16)][0]
                nb = (n + 127) // 128

                def ring(o, carry):
                    for q in range(NSLOT):
                        b = o * NSLOT + q

                        @pl.when(b < nb)
                        def _(b=b, q=q):
                            @pl.when(b >= NSLOT)
                            def _():
                                pltpu.make_async_copy(
                                    dummy, rows[q], ssem[q]).wait()
                            pltpu.async_copy(
                                pp.at[sidx_v.at[j, b]], rows[q], gsem[q])

                    for q in range(NSLOT):
                        b = o * NSLOT + q

                        @pl.when(b < nb)
                        def _(b=b, q=q):
                            pltpu.make_async_copy(
                                dummy, rows[q], gsem[q]).wait()
                            pltpu.async_copy(
                                rows[q], acc_sh.at[didx_v.at[j, b]],
                                ssem[q], add=True)

                    return carry

                lax.fori_loop(0, (nb + NSLOT - 1) // NSLOT, ring, 0)
                for q in range(NSLOT):
                    @pl.when(q < nb)
                    def _(q=q):
                        pltpu.make_async_copy(dummy, rows[q], ssem[q]).wait()
            plsc.subcore_barrier()
            sl = pl.ds(s * 320, 320)
            pltpu.sync_copy(acc_sh.at[sl], out.at[pl.ds(cbase + s * 320, 320)])
            plsc.subcore_barrier()

    @pl.when(c == 0)
    def _():
        run_half(0, 0)

    @pl.when(c == 1)
    def _():
        run_half(1, HALF)


@functools.cache
def _agg_call():
    return pl.kernel(
        _agg_body,
        out_type=jax.ShapeDtypeStruct((NCH, NP, 128), jnp.float32),
        mesh=_mesh(),
        scratch_types=[
            pltpu.VMEM((2, CAP // 128, 128), jnp.int32),
            pltpu.VMEM((2, CAP // 128, 128), jnp.int32),
            pltpu.VMEM((256,), jnp.int32),
            pltpu.VMEM((128, 128), jnp.float32),
            pltpu.VMEM((128, 128), jnp.float32),
            pltpu.VMEM((128, 128), jnp.float32),
            pltpu.VMEM((64, 128), jnp.float32),
            pltpu.VMEM_SHARED((ACCR, 128), jnp.float32),
            pltpu.SemaphoreType.DMA,
            pltpu.SemaphoreType.DMA,
            pltpu.SemaphoreType.DMA,
            pltpu.SemaphoreType.DMA,
            pltpu.SemaphoreType.DMA,
            pltpu.SemaphoreType.DMA,
        ],
    )


def _dinv_body(dp_ref, out_ref):
    out_ref[...] = lax.rsqrt(jnp.sum(dp_ref[...], axis=0) + 1.0)


def _dinv_call(dp):
    return pl.pallas_call(
        _dinv_body,
        out_shape=jax.ShapeDtypeStruct((NP // 128, 128), jnp.float32),
    )(dp)


def _w_body(sp_ref, dinv_ref, out_ref):
    ssum = jnp.sum(sp_ref[...], axis=0)
    dinv = dinv_ref[...]
    r = lax.broadcasted_iota(jnp.int32, (NP // 128, 128), 0)
    l = lax.broadcasted_iota(jnp.int32, (NP // 128, 128), 1)
    valid = (r * 128 + l) < N
    w = dinv * (ssum + dinv) * (1.0 / N)
    out_ref[...] = jnp.where(valid, w, 0.0)


def _w_call(sp, dinv):
    return pl.pallas_call(
        _w_body,
        out_shape=jax.ShapeDtypeStruct((NP // 128, 128), jnp.float32),
    )(sp, dinv)


def _mm_body(h_ref, wcat_ref, dvr_ref, lb_ref, pp_ref, r_ref):
    prod = jnp.dot(h_ref[...], wcat_ref[...], preferred_element_type=jnp.float32)
    dvr = dvr_ref[...]
    for f in range(NCH):
        pp_ref[f] = prod[:, 128 * f:128 * (f + 1)] * dvr
    r_ref[...] = prod[:, DH:] + lb_ref[...].reshape(1, DH)


def _mm_call(h, wcat, dvr, lb):
    k = h.shape[1]
    return pl.pallas_call(
        _mm_body,
        grid=(GR,),
        in_specs=[
            pl.BlockSpec((RB, k), lambda i: (i, 0)),
            pl.BlockSpec((k, DH * 2), lambda i: (0, 0)),
            pl.BlockSpec((RB, 128), lambda i: (i, 0)),
            pl.BlockSpec((1, 1, DH), lambda i: (0, 0, 0)),
        ],
        out_specs=(
            pl.BlockSpec((NCH, RB, 128), lambda i: (0, i, 0)),
            pl.BlockSpec((RB, DH), lambda i: (i, 0)),
        ),
        out_shape=(
            jax.ShapeDtypeStruct((NCH, NP, 128), jnp.float32),
            jax.ShapeDtypeStruct((NP, DH), jnp.float32),
        ),
        compiler_params=pltpu.CompilerParams(dimension_semantics=("arbitrary",)),
    )(h, wcat, dvr, lb)


def _bn_body(sc_ref, pp_ref, dvr_ref, r_ref, g_ref, be_ref, wrep_ref,
             h_ref, wsum_ref, stats, *, last):
    p = pl.program_id(0)
    i = pl.program_id(1)
    dvr = dvr_ref[...]
    y = jnp.concatenate(
        [(sc_ref[f] + pp_ref[f]) * dvr for f in range(NCH)], axis=1)

    @pl.when(jnp.logical_and(p == 0, i == 0))
    def _():
        stats[...] = jnp.zeros_like(stats)

    @pl.when(p == 0)
    def _():
        rows = i * RB + lax.broadcasted_iota(jnp.int32, (RB, DH), 0)
        ym = jnp.where(rows < N, y, 0.0)
        stats[0:1, :] = stats[0:1, :] + jnp.sum(ym, axis=0, keepdims=True)
        stats[1:2, :] = stats[1:2, :] + jnp.sum(ym * ym, axis=0, keepdims=True)

    @pl.when(p == 1)
    def _():
        m = stats[0:1, :] * (1.0 / N)
        v = stats[1:2, :] * (1.0 / N) - m * m
        rstd = lax.rsqrt(v + EPS)
        g = g_ref[...].reshape(1, DH)
        be = be_ref[...].reshape(1, DH)
        hn = jnp.maximum((y - m) * rstd * g + be, 0.0) + r_ref[...]
        h_ref[...] = hn
        if last:
            wr = wrep_ref[...]
            wcat = jnp.concatenate([wr, wr, wr, wr], axis=1)
            stats[2:3, :] = stats[2:3, :] + jnp.sum(hn * wcat, axis=0, keepdims=True)

            @pl.when(i == GR - 1)
            def _():
                wsum_ref[...] = stats[...]


def _bn_call(scs, pps, dvr, r, g, be, wrep, *, last):
    def body(sc_ref, pp_ref, dv, rr, gg, bb, *rest):
        if last:
            (wrp, h_ref, wsum_ref, stats) = rest
        else:
            (h_ref, stats) = rest
            wrp, wsum_ref = None, None
        _bn_body(sc_ref, pp_ref, dv, rr, gg, bb, wrp,
                 h_ref, wsum_ref, stats, last=last)

    chunk_spec = pl.BlockSpec((RB, 128), lambda p, i: (i, 0))
    stack_spec = pl.BlockSpec((NCH, RB, 128), lambda p, i: (0, i, 0))
    in_specs = (
        [stack_spec, stack_spec]
        + [chunk_spec,
           pl.BlockSpec((RB, DH), lambda p, i: (i, 0)),
           pl.BlockSpec((1, 1, DH), lambda p, i: (0, 0, 0)),
           pl.BlockSpec((1, 1, DH), lambda p, i: (0, 0, 0))]
    )
    out_specs = [pl.BlockSpec((RB, DH), lambda p, i: (p * i, 0))]
    out_shape = [jax.ShapeDtypeStruct((NP, DH), jnp.float32)]
    args = [scs, pps, dvr, r, g, be]
    if last:
        in_specs.append(chunk_spec)
        args.append(wrep)
        out_specs.append(pl.BlockSpec((8, DH), lambda p, i: (0, 0)))
        out_shape.append(jax.ShapeDtypeStruct((8, DH), jnp.float32))
    res = pl.pallas_call(
        body,
        grid=(2, GR),
        in_specs=in_specs,
        out_specs=tuple(out_specs),
        out_shape=tuple(out_shape),
        scratch_shapes=[pltpu.VMEM((8, DH), jnp.float32)],
        compiler_params=pltpu.CompilerParams(
            dimension_semantics=("arbitrary", "arbitrary")),
    )(*args)
    return res if last else (res[0], None)


def _fin_body(ws_ref, w3_ref, b3_ref, out_ref):
    v = ws_ref[2:3, :]
    res = jnp.dot(v, w3_ref[...], preferred_element_type=jnp.float32)
    res = res + b3_ref[...].reshape(1, DOUT)
    out_ref[...] = jnp.broadcast_to(res, (8, DOUT))


def _fin_call(wsum, W3, b3):
    return pl.pallas_call(
        _fin_body,
        out_shape=jax.ShapeDtypeStruct((8, DOUT), jnp.float32),
    )(wsum, W3, b3)


def kernel(x, edge_index, W0, b0, W1, b1, W2, b2, W3, b3,
           g0, be0, g1, be1, g2, be2, Lw0, Lb0, Lw1, Lb1, Lw2, Lb2):
    src = edge_index[0]
    dst = edge_index[1]
    src_w = src.reshape(NC, NS, EPW)
    dst_w = dst.reshape(NC, NS, EPW)
    idx_w = edge_index.reshape(2, NC, NS, EPW)
    psrc, pdst, cnts = _part_call()(idx_w)
    psrc = psrc.reshape(2, NW, CAP // 128, 128)
    pdst = pdst.reshape(2, NW, CAP // 128, 128)

    deg_parts = _deg_call()(dst_w)
    dinv2d = _dinv_call(deg_parts.reshape(NC * NS, NP // 128, 128))
    dinv_lin = dinv2d.reshape(NP)
    dvr = jnp.broadcast_to(dinv_lin[:, None], (NP, 128))
    s_parts = _s_call()(src_w, dst_w, dinv_lin)
    w2d = _w_call(s_parts.reshape(NC * NS, NP // 128, 128), dinv2d)
    wrep = jnp.broadcast_to(w2d.reshape(NP)[:, None], (NP, 128))

    h = jnp.pad(x, ((0, NP - N), (0, 0)))
    layer_params = ((W0, Lw0, Lb0, g0, be0),
                    (W1, Lw1, Lb1, g1, be1),
                    (W2, Lw2, Lb2, g2, be2))
    wsum = None
    for li, (W, Lw, Lb, g, be) in enumerate(layer_params):
        wcat = jnp.concatenate([W, Lw], axis=1)
        pps, r = _mm_call(h, wcat, dvr, Lb.reshape(1, 1, DH))
        scs = _agg_call()(pps, psrc, pdst, cnts)
        h, wsum = _bn_call(scs, pps, dvr, r, g.reshape(1, 1, DH),
                           be.reshape(1, 1, DH), wrep, last=(li == 2))

    out = _fin_call(wsum, W3, b3.reshape(1, 1, DOUT))
    return out[0:1]


# speedup vs baseline: 7.9560x; 1.6312x over previous
import functools

import jax
import jax.numpy as jnp
from jax import lax
from jax.experimental import pallas as pl
from jax.experimental.pallas import tpu as pltpu
from jax.experimental.pallas import tpu_sc as plsc

N = 10000
NP = 10240
E = 160000
DIN = 256
DH = 512
DOUT = 256
EPS = 1e-5
NC = 2
NS = 16
EPW = E // (NC * NS)
NW = NC * NS
HALF = NP // 2
CAP = 5248
ACCR = HALF + 128
RB = 512
GR = NP // RB
NCH = 4
QTR = NP // 4
NBKT = 8

@functools.cache
def _mesh():
    return plsc.VectorSubcoreMesh(core_axis_name="c", subcore_axis_name="s",
                                  num_cores=NC, num_subcores=NS)


def _zero_hist(hist_v, nwords):
    def zero(i, carry):
        hist_v[pl.ds(i * 16, 16)] = jnp.zeros((16,), jnp.float32)
        return carry

    lax.fori_loop(0, nwords // 16, zero, 0)


def _deg_body(dst_hbm, out_hbm, idx_v, hist_v):
    c = lax.axis_index("c")
    s = lax.axis_index("s")
    pltpu.sync_copy(dst_hbm.at[c, s], idx_v)
    _zero_hist(hist_v, NP + 16)
    one0 = jnp.where(lax.iota(jnp.int32, 16) == 0, 1.0, 0.0)

    def body(i, carry):
        dv = idx_v[pl.ds(i * 16, 16)]
        for k in range(16):
            d = dv[k]
            hist_v[pl.ds(d, 16)] = hist_v[pl.ds(d, 16)] + one0
        return carry

    lax.fori_loop(0, EPW // 16, body, 0)
    pltpu.sync_copy(hist_v.at[pl.ds(0, NP)], out_hbm.at[c, s])


@functools.cache
def _deg_call():
    return pl.kernel(
        _deg_body,
        out_type=jax.ShapeDtypeStruct((NC, NS, NP), jnp.float32),
        mesh=_mesh(),
        scratch_types=[
            pltpu.VMEM((EPW,), jnp.int32),
            pltpu.VMEM((NP + 16,), jnp.float32),
        ],
    )


def _s_body(src_hbm, dst_hbm, dinv_hbm, out_hbm, sidx_v, didx_v, dinv_v, hist_v):
    c = lax.axis_index("c")
    s = lax.axis_index("s")
    pltpu.sync_copy(src_hbm.at[c, s], sidx_v)
    pltpu.sync_copy(dst_hbm.at[c, s], didx_v)
    pltpu.sync_copy(dinv_hbm, dinv_v.at[pl.ds(0, NP)])
    _zero_hist(hist_v, NP + 16)
    io16 = lax.iota(jnp.int32, 16)

    def body(i, carry):
        sv = sidx_v[pl.ds(i * 16, 16)]
        dv = didx_v[pl.ds(i * 16, 16)]
        for k in range(16):
            s0 = sv[k]
            dval = dinv_v[pl.ds(dv[k], 16)][0]
            upd = jnp.where(io16 == 0, dval, 0.0)
            hist_v[pl.ds(s0, 16)] = hist_v[pl.ds(s0, 16)] + upd
        return carry

    lax.fori_loop(0, EPW // 16, body, 0)
    pltpu.sync_copy(hist_v.at[pl.ds(0, NP)], out_hbm.at[c, s])


@functools.cache
def _s_call():
    return pl.kernel(
        _s_body,
        out_type=jax.ShapeDtypeStruct((NC, NS, NP), jnp.float32),
        mesh=_mesh(),
        scratch_types=[
            pltpu.VMEM((EPW,), jnp.int32),
            pltpu.VMEM((EPW,), jnp.int32),
            pltpu.VMEM((NP + 16,), jnp.float32),
            pltpu.VMEM((NP + 16,), jnp.float32),
        ],
    )


def _part_body(idx_hbm, psrc_hbm, pdst_hbm, cnt_hbm,
               sidx_v, didx_v, bsrc_v, bdst_v, offs_v, cnt_v):
    c = lax.axis_index("c")
    s = lax.axis_index("s")
    w = c * NS + s
    pltpu.sync_copy(idx_hbm.at[0, c, s], sidx_v)
    pltpu.sync_copy(idx_hbm.at[1, c, s], didx_v)
    io16 = lax.iota(jnp.int32, 16)
    offs_v[pl.ds(0, 16)] = jnp.zeros((16,), jnp.int32)
    offs_v[pl.ds(16, 16)] = jnp.zeros((16,), jnp.int32)

    def body(i, carry):
        sv = sidx_v[pl.ds(i * 16, 16)]
        dv = didx_v[pl.ds(i * 16, 16)]
        for k in range(16):
            sk = sv[k]
            dk = dv[k]
            hi = (dk >= HALF).astype(jnp.int32)
            q = ((sk >= QTR).astype(jnp.int32)
                 + (sk >= 2 * QTR).astype(jnp.int32)
                 + (sk >= 3 * QTR).astype(jnp.int32))
            g = 4 * hi + q
            sl = sk - q * QTR
            dl = dk - hi * HALF
            ov = offs_v[pl.ds(g, 16)]
            off = ov[0]
            pos = g * CAP + off
            bsrc_v[pl.ds(pos, 16)] = jnp.where(io16 == 0, sl, 0)
            bdst_v[pl.ds(pos, 16)] = jnp.where(io16 == 0, dl, 0)
            offs_v[pl.ds(g, 16)] = jnp.where(io16 == 0, off + 1, ov)
        return carry

    lax.fori_loop(0, EPW // 16, body, 0)
    zsrc = jnp.zeros((16,), jnp.int32)

    def czero(i, carry):
        cnt_v[pl.ds(i * 16, 16)] = jnp.zeros((16,), jnp.int32)
        return carry

    lax.fori_loop(0, NBKT * 128 // 16, czero, 0)
    for g in range(NBKT):
        off = offs_v[pl.ds(g, 16)][0]
        for t in range(4):
            trash = HALF + io16 + 16 * t
            bsrc_v[pl.ds(g * CAP + off + t * 16, 16)] = zsrc
            bdst_v[pl.ds(g * CAP + off + t * 16, 16)] = trash
        cnt_v[pl.ds(g * 128, 16)] = jnp.where(io16 == 0, off, 0)
    pltpu.sync_copy(bsrc_v.at[pl.ds(0, NBKT * CAP)], psrc_hbm.at[w])
    pltpu.sync_copy(bdst_v.at[pl.ds(0, NBKT * CAP)], pdst_hbm.at[w])
    pltpu.sync_copy(cnt_v, cnt_hbm.at[w])


@functools.cache
def _part_call():
    return pl.kernel(
        _part_body,
        out_type=(
            jax.ShapeDtypeStruct((NW, NBKT * CAP), jnp.int32),
            jax.ShapeDtypeStruct((NW, NBKT * CAP), jnp.int32),
            jax.ShapeDtypeStruct((NW, NBKT * 128), jnp.int32),
        ),
        mesh=_mesh(),
        scratch_types=[
            pltpu.VMEM((EPW,), jnp.int32),
            pltpu.VMEM((EPW,), jnp.int32),
            pltpu.VMEM((NBKT * CAP + 16,), jnp.int32),
            pltpu.VMEM((NBKT * CAP + 16,), jnp.int32),
            pltpu.VMEM((32,), jnp.int32),
            pltpu.VMEM((NBKT * 128,), jnp.int32),
        ],
    )


NSLOT = 3
SUBW = 64


def _agg_body(pp_hbm, psrc_hbm, pdst_hbm, cnt_hbm, out_hbm,
              sidx_v, didx_v, cnt_v, r0, r1, r2, zbuf_v, tbl_sh, acc_sh,
              g0, g1, g2, s0, s1, s2):
    c = lax.axis_index("c")
    s = lax.axis_index("s")
    rows = (r0, r1, r2)
    gsem = (g0, g1, g2)
    ssem = (s0, s1, s2)

    def zfill(r, carry):
        for k in range(8):
            zbuf_v[r, pl.ds(k * 16, 16)] = jnp.zeros((16,), jnp.float32)
        return carry

    lax.fori_loop(0, 32, zfill, 0)

    cbase = c * HALF
    pltpu.sync_copy(cnt_hbm.at[2 * s], cnt_v.at[pl.ds(0, NBKT * 128)])
    pltpu.sync_copy(cnt_hbm.at[2 * s + 1],
                    cnt_v.at[pl.ds(NBKT * 128, NBKT * 128)])

    def chunk_loop(f, carry0):
        pp = pp_hbm.at[f]
        dummy = pp.at[pl.ds(0, 64)]
        for z in range(10):
            pltpu.sync_copy(zbuf_v, acc_sh.at[pl.ds(s * 328 + z * 32, 32)])
        pltpu.sync_copy(zbuf_v.at[pl.ds(0, 8)],
                        acc_sh.at[pl.ds(s * 328 + 320, 8)])

        def quarter_loop(sg, carry1):
            g = 4 * c + sg
            pltpu.sync_copy(pp.at[pl.ds(sg * QTR + s * 160, 160)],
                            tbl_sh.at[pl.ds(s * 160, 160)])
            plsc.subcore_barrier()
            for j in range(2):
                w = 2 * s + j
                pltpu.sync_copy(psrc_hbm.at[w, g], sidx_v)
                pltpu.sync_copy(pdst_hbm.at[w, g], didx_v)
                n = cnt_v[pl.ds(j * NBKT * 128 + g * 128, 16)][0]
                nb = (n + 63) // 64

                def ring(o, carry2):
                    for q in range(NSLOT):
                        b = o * NSLOT + q

                        @pl.when(b < nb)
                        def _(b=b, q=q):
                            @pl.when(b >= NSLOT)
                            def _():
                                pltpu.make_async_copy(
                                    dummy, rows[q], ssem[q]).wait()
                            pltpu.async_copy(
                                tbl_sh.at[sidx_v.at[b]], rows[q], gsem[q])

                    for q in range(NSLOT):
                        b = o * NSLOT + q

                        @pl.when(b < nb)
                        def _(b=b, q=q):
                            pltpu.make_async_copy(
                                dummy, rows[q], gsem[q]).wait()
                            pltpu.async_copy(
                                rows[q], acc_sh.at[didx_v.at[b]],
                                ssem[q], add=True)

                    return carry2

                lax.fori_loop(0, (nb + NSLOT - 1) // NSLOT, ring, 0)
                for q in range(NSLOT):
                    @pl.when(q < nb)
                    def _(q=q):
                        pltpu.make_async_copy(dummy, rows[q],
                                              ssem[q]).wait()
            plsc.subcore_barrier()
            return carry1

        lax.fori_loop(0, 4, quarter_loop, 0)
        pltpu.sync_copy(
            acc_sh.at[pl.ds(s * 320, 320)],
            out_hbm.at[f].at[pl.ds(cbase + s * 320, 320)])
        plsc.subcore_barrier()
        return carry0

    lax.fori_loop(0, NCH, chunk_loop, 0)


@functools.cache
def _agg_call():
    return pl.kernel(
        _agg_body,
        out_type=jax.ShapeDtypeStruct((NCH, NP, 128), jnp.float32),
        mesh=_mesh(),
        scratch_types=[
            pltpu.VMEM((CAP // 64, 64), jnp.int32),
            pltpu.VMEM((CAP // 64, 64), jnp.int32),
            pltpu.VMEM((2 * NBKT * 128,), jnp.int32),
            pltpu.VMEM((64, 128), jnp.float32),
            pltpu.VMEM((64, 128), jnp.float32),
            pltpu.VMEM((64, 128), jnp.float32),
            pltpu.VMEM((32, 128), jnp.float32),
            pltpu.VMEM_SHARED((QTR, 128), jnp.float32),
            pltpu.VMEM_SHARED((ACCR, 128), jnp.float32),
            pltpu.SemaphoreType.DMA,
            pltpu.SemaphoreType.DMA,
            pltpu.SemaphoreType.DMA,
            pltpu.SemaphoreType.DMA,
            pltpu.SemaphoreType.DMA,
            pltpu.SemaphoreType.DMA,
        ],
    )


def _dinv_body(dp_ref, out_ref):
    out_ref[...] = lax.rsqrt(jnp.sum(dp_ref[...], axis=0) + 1.0)


def _dinv_call(dp):
    return pl.pallas_call(
        _dinv_body,
        out_shape=jax.ShapeDtypeStruct((NP // 128, 128), jnp.float32),
    )(dp)


def _w_body(sp_ref, dinv_ref, out_ref):
    ssum = jnp.sum(sp_ref[...], axis=0)
    dinv = dinv_ref[...]
    r = lax.broadcasted_iota(jnp.int32, (NP // 128, 128), 0)
    l = lax.broadcasted_iota(jnp.int32, (NP // 128, 128), 1)
    valid = (r * 128 + l) < N
    w = dinv * (ssum + dinv) * (1.0 / N)
    out_ref[...] = jnp.where(valid, w, 0.0)


def _w_call(sp, dinv):
    return pl.pallas_call(
        _w_body,
        out_shape=jax.ShapeDtypeStruct((NP // 128, 128), jnp.float32),
    )(sp, dinv)


def _mm_body(h_ref, wcat_ref, dvr_ref, lb_ref, pp_ref, r_ref):
    prod = jnp.dot(h_ref[...], wcat_ref[...], preferred_element_type=jnp.float32)
    dvr = dvr_ref[...]
    for f in range(NCH):
        pp_ref[f] = prod[:, 128 * f:128 * (f + 1)] * dvr
    r_ref[...] = prod[:, DH:] + lb_ref[...].reshape(1, DH)


def _mm_call(h, wcat, dvr, lb):
    k = h.shape[1]
    return pl.pallas_call(
        _mm_body,
        grid=(GR,),
        in_specs=[
            pl.BlockSpec((RB, k), lambda i: (i, 0)),
            pl.BlockSpec((k, DH * 2), lambda i: (0, 0)),
            pl.BlockSpec((RB, 128), lambda i: (i, 0)),
            pl.BlockSpec((1, 1, DH), lambda i: (0, 0, 0)),
        ],
        out_specs=(
            pl.BlockSpec((NCH, RB, 128), lambda i: (0, i, 0)),
            pl.BlockSpec((RB, DH), lambda i: (i, 0)),
        ),
        out_shape=(
            jax.ShapeDtypeStruct((NCH, NP, 128), jnp.float32),
            jax.ShapeDtypeStruct((NP, DH), jnp.float32),
        ),
        compiler_params=pltpu.CompilerParams(dimension_semantics=("arbitrary",)),
    )(h, wcat, dvr, lb)


def _bn_body(sc_ref, pp_ref, dvr_ref, r_ref, g_ref, be_ref, wrep_ref,
             h_ref, wsum_ref, stats, *, last):
    p = pl.program_id(0)
    i = pl.program_id(1)
    dvr = dvr_ref[...]
    y = jnp.concatenate(
        [(sc_ref[f] + pp_ref[f]) * dvr for f in range(NCH)], axis=1)

    @pl.when(jnp.logical_and(p == 0, i == 0))
    def _():
        stats[...] = jnp.zeros_like(stats)

    @pl.when(p == 0)
    def _():
        rows = i * RB + lax.broadcasted_iota(jnp.int32, (RB, DH), 0)
        ym = jnp.where(rows < N, y, 0.0)
        stats[0:1, :] = stats[0:1, :] + jnp.sum(ym, axis=0, keepdims=True)
        stats[1:2, :] = stats[1:2, :] + jnp.sum(ym * ym, axis=0, keepdims=True)

    @pl.when(p == 1)
    def _():
        m = stats[0:1, :] * (1.0 / N)
        v = stats[1:2, :] * (1.0 / N) - m * m
        rstd = lax.rsqrt(v + EPS)
        g = g_ref[...].reshape(1, DH)
        be = be_ref[...].reshape(1, DH)
        hn = jnp.maximum((y - m) * rstd * g + be, 0.0) + r_ref[...]
        h_ref[...] = hn
        if last:
            wr = wrep_ref[...]
            wcat = jnp.concatenate([wr, wr, wr, wr], axis=1)
            stats[2:3, :] = stats[2:3, :] + jnp.sum(hn * wcat, axis=0, keepdims=True)

            @pl.when(i == GR - 1)
            def _():
                wsum_ref[...] = stats[...]


def _bn_call(scs, pps, dvr, r, g, be, wrep, *, last):
    def body(sc_ref, pp_ref, dv, rr, gg, bb, *rest):
        if last:
            (wrp, h_ref, wsum_ref, stats) = rest
        else:
            (h_ref, stats) = rest
            wrp, wsum_ref = None, None
        _bn_body(sc_ref, pp_ref, dv, rr, gg, bb, wrp,
                 h_ref, wsum_ref, stats, last=last)

    chunk_spec = pl.BlockSpec((RB, 128), lambda p, i: (i, 0))
    stack_spec = pl.BlockSpec((NCH, RB, 128), lambda p, i: (0, i, 0))
    in_specs = (
        [stack_spec, stack_spec]
        + [chunk_spec,
           pl.BlockSpec((RB, DH), lambda p, i: (i, 0)),
           pl.BlockSpec((1, 1, DH), lambda p, i: (0, 0, 0)),
           pl.BlockSpec((1, 1, DH), lambda p, i: (0, 0, 0))]
    )
    out_specs = [pl.BlockSpec((RB, DH), lambda p, i: (p * i, 0))]
    out_shape = [jax.ShapeDtypeStruct((NP, DH), jnp.float32)]
    args = [scs, pps, dvr, r, g, be]
    if last:
        in_specs.append(chunk_spec)
        args.append(wrep)
        out_specs.append(pl.BlockSpec((8, DH), lambda p, i: (0, 0)))
        out_shape.append(jax.ShapeDtypeStruct((8, DH), jnp.float32))
    res = pl.pallas_call(
        body,
        grid=(2, GR),
        in_specs=in_specs,
        out_specs=tuple(out_specs),
        out_shape=tuple(out_shape),
        scratch_shapes=[pltpu.VMEM((8, DH), jnp.float32)],
        compiler_params=pltpu.CompilerParams(
            dimension_semantics=("arbitrary", "arbitrary")),
    )(*args)
    return res if last else (res[0], None)


def _fin_body(ws_ref, w3_ref, b3_ref, out_ref):
    v = ws_ref[2:3, :]
    res = jnp.dot(v, w3_ref[...], preferred_element_type=jnp.float32)
    res = res + b3_ref[...].reshape(1, DOUT)
    out_ref[...] = jnp.broadcast_to(res, (8, DOUT))


def _fin_call(wsum, W3, b3):
    return pl.pallas_call(
        _fin_body,
        out_shape=jax.ShapeDtypeStruct((8, DOUT), jnp.float32),
    )(wsum, W3, b3)


def kernel(x, edge_index, W0, b0, W1, b1, W2, b2, W3, b3,
           g0, be0, g1, be1, g2, be2, Lw0, Lb0, Lw1, Lb1, Lw2, Lb2):
    src = edge_index[0]
    dst = edge_index[1]
    src_w = src.reshape(NC, NS, EPW)
    dst_w = dst.reshape(NC, NS, EPW)
    idx_w = edge_index.reshape(2, NC, NS, EPW)
    psrc, pdst, cnts = _part_call()(idx_w)
    psrc = psrc.reshape(NW, NBKT, CAP // 64, 64)
    pdst = pdst.reshape(NW, NBKT, CAP // 64, 64)

    deg_parts = _deg_call()(dst_w)
    dinv2d = _dinv_call(deg_parts.reshape(NC * NS, NP // 128, 128))
    dinv_lin = dinv2d.reshape(NP)
    dvr = jnp.broadcast_to(dinv_lin[:, None], (NP, 128))
    s_parts = _s_call()(src_w, dst_w, dinv_lin)
    w2d = _w_call(s_parts.reshape(NC * NS, NP // 128, 128), dinv2d)
    wrep = jnp.broadcast_to(w2d.reshape(NP)[:, None], (NP, 128))

    h = jnp.pad(x, ((0, NP - N), (0, 0)))
    layer_params = ((W0, Lw0, Lb0, g0, be0),
                    (W1, Lw1, Lb1, g1, be1),
                    (W2, Lw2, Lb2, g2, be2))
    wsum = None
    for li, (W, Lw, Lb, g, be) in enumerate(layer_params):
        wcat = jnp.concatenate([W, Lw], axis=1)
        pps, r = _mm_call(h, wcat, dvr, Lb.reshape(1, 1, DH))
        scs = _agg_call()(pps, psrc, pdst, cnts)
        h, wsum = _bn_call(scs, pps, dvr, r, g.reshape(1, 1, DH),
                           be.reshape(1, 1, DH), wrep, last=(li == 2))

    out = _fin_call(wsum, W3, b3.reshape(1, 1, DOUT))
    return out[0:1]


# speedup vs baseline: 8.3961x; 1.0553x over previous
import functools

import jax
import jax.numpy as jnp
from jax import lax
from jax.experimental import pallas as pl
from jax.experimental.pallas import tpu as pltpu
from jax.experimental.pallas import tpu_sc as plsc

N = 10000
NP = 10240
E = 160000
DIN = 256
DH = 512
DOUT = 256
EPS = 1e-5
NC = 2
NS = 16
EPW = E // (NC * NS)
NW = NC * NS
HALF = NP // 2
CAP = 5248
ACCR = HALF + 128
RB = 512
GR = NP // RB
NCH = 4
QTR = NP // 4
NBKT = 8

@functools.cache
def _mesh():
    return plsc.VectorSubcoreMesh(core_axis_name="c", subcore_axis_name="s",
                                  num_cores=NC, num_subcores=NS)


def _zero_hist(hist_v, nwords):
    def zero(i, carry):
        hist_v[pl.ds(i * 16, 16)] = jnp.zeros((16,), jnp.float32)
        return carry

    lax.fori_loop(0, nwords // 16, zero, 0)


def _deg_body(dst_hbm, out_hbm, idx_v, hist_v):
    c = lax.axis_index("c")
    s = lax.axis_index("s")
    pltpu.sync_copy(dst_hbm.at[c, s], idx_v)
    _zero_hist(hist_v, NP + 16)
    one0 = jnp.where(lax.iota(jnp.int32, 16) == 0, 1.0, 0.0)

    def body(i, carry):
        dv = idx_v[pl.ds(i * 16, 16)]
        for k in range(16):
            d = dv[k]
            hist_v[pl.ds(d, 16)] = hist_v[pl.ds(d, 16)] + one0
        return carry

    lax.fori_loop(0, EPW // 16, body, 0)
    pltpu.sync_copy(hist_v.at[pl.ds(0, NP)], out_hbm.at[c, s])


@functools.cache
def _deg_call():
    return pl.kernel(
        _deg_body,
        out_type=jax.ShapeDtypeStruct((NC, NS, NP), jnp.float32),
        mesh=_mesh(),
        scratch_types=[
            pltpu.VMEM((EPW,), jnp.int32),
            pltpu.VMEM((NP + 16,), jnp.float32),
        ],
    )


def _s_body(src_hbm, dst_hbm, dinv_hbm, out_hbm, sidx_v, didx_v, dinv_v, hist_v):
    c = lax.axis_index("c")
    s = lax.axis_index("s")
    pltpu.sync_copy(src_hbm.at[c, s], sidx_v)
    pltpu.sync_copy(dst_hbm.at[c, s], didx_v)
    pltpu.sync_copy(dinv_hbm, dinv_v.at[pl.ds(0, NP)])
    _zero_hist(hist_v, NP + 16)
    io16 = lax.iota(jnp.int32, 16)

    def body(i, carry):
        sv = sidx_v[pl.ds(i * 16, 16)]
        dv = didx_v[pl.ds(i * 16, 16)]
        for k in range(16):
            s0 = sv[k]
            dval = dinv_v[pl.ds(dv[k], 16)][0]
            upd = jnp.where(io16 == 0, dval, 0.0)
            hist_v[pl.ds(s0, 16)] = hist_v[pl.ds(s0, 16)] + upd
        return carry

    lax.fori_loop(0, EPW // 16, body, 0)
    pltpu.sync_copy(hist_v.at[pl.ds(0, NP)], out_hbm.at[c, s])


@functools.cache
def _s_call():
    return pl.kernel(
        _s_body,
        out_type=jax.ShapeDtypeStruct((NC, NS, NP), jnp.float32),
        mesh=_mesh(),
        scratch_types=[
            pltpu.VMEM((EPW,), jnp.int32),
            pltpu.VMEM((EPW,), jnp.int32),
            pltpu.VMEM((NP + 16,), jnp.float32),
            pltpu.VMEM((NP + 16,), jnp.float32),
        ],
    )


def _part_body(idx_hbm, psrc_hbm, pdst_hbm, cnt_hbm,
               sidx_v, didx_v, bsrc_v, bdst_v, offs_v, cnt_v):
    c = lax.axis_index("c")
    s = lax.axis_index("s")
    w = c * NS + s
    pltpu.sync_copy(idx_hbm.at[0, c, s], sidx_v)
    pltpu.sync_copy(idx_hbm.at[1, c, s], didx_v)
    io16 = lax.iota(jnp.int32, 16)
    offs_v[pl.ds(0, 16)] = jnp.zeros((16,), jnp.int32)
    offs_v[pl.ds(16, 16)] = jnp.zeros((16,), jnp.int32)

    def body(i, carry):
        sv = sidx_v[pl.ds(i * 16, 16)]
        dv = didx_v[pl.ds(i * 16, 16)]
        for k in range(16):
            sk = sv[k]
            dk = dv[k]
            hi = (dk >= HALF).astype(jnp.int32)
            q = ((sk >= QTR).astype(jnp.int32)
                 + (sk >= 2 * QTR).astype(jnp.int32)
                 + (sk >= 3 * QTR).astype(jnp.int32))
            g = 4 * hi + q
            sl = sk - q * QTR
            dl = dk - hi * HALF
            ov = offs_v[pl.ds(g, 16)]
            off = ov[0]
            pos = g * CAP + off
            bsrc_v[pl.ds(pos, 16)] = jnp.where(io16 == 0, sl, 0)
            bdst_v[pl.ds(pos, 16)] = jnp.where(io16 == 0, dl, 0)
            offs_v[pl.ds(g, 16)] = jnp.where(io16 == 0, off + 1, ov)
        return carry

    lax.fori_loop(0, EPW // 16, body, 0)
    zsrc = jnp.zeros((16,), jnp.int32)

    def czero(i, carry):
        cnt_v[pl.ds(i * 16, 16)] = jnp.zeros((16,), jnp.int32)
        return carry

    lax.fori_loop(0, NBKT * 128 // 16, czero, 0)
    for g in range(NBKT):
        off = offs_v[pl.ds(g, 16)][0]
        for t in range(4):
            trash = HALF + io16 + 16 * t
            bsrc_v[pl.ds(g * CAP + off + t * 16, 16)] = zsrc
            bdst_v[pl.ds(g * CAP + off + t * 16, 16)] = trash
        cnt_v[pl.ds(g * 128, 16)] = jnp.where(io16 == 0, off, 0)
    pltpu.sync_copy(bsrc_v.at[pl.ds(0, NBKT * CAP)], psrc_hbm.at[w])
    pltpu.sync_copy(bdst_v.at[pl.ds(0, NBKT * CAP)], pdst_hbm.at[w])
    pltpu.sync_copy(cnt_v, cnt_hbm.at[w])


@functools.cache
def _part_call():
    return pl.kernel(
        _part_body,
        out_type=(
            jax.ShapeDtypeStruct((NW, NBKT * CAP), jnp.int32),
            jax.ShapeDtypeStruct((NW, NBKT * CAP), jnp.int32),
            jax.ShapeDtypeStruct((NW, NBKT * 128), jnp.int32),
        ),
        mesh=_mesh(),
        scratch_types=[
            pltpu.VMEM((EPW,), jnp.int32),
            pltpu.VMEM((EPW,), jnp.int32),
            pltpu.VMEM((NBKT * CAP + 16,), jnp.int32),
            pltpu.VMEM((NBKT * CAP + 16,), jnp.int32),
            pltpu.VMEM((32,), jnp.int32),
            pltpu.VMEM((NBKT * 128,), jnp.int32),
        ],
    )


NSLOT = 4
SUBW = 64


def _agg_body(pp_hbm, psrc_hbm, pdst_hbm, cnt_hbm, out_hbm,
              sidx_v, didx_v, cnt_v, r0, r1, r2, r3, zbuf_v, tbl_sh, acc_sh,
              g0, g1, g2, g3, s0, s1, s2, s3):
    c = lax.axis_index("c")
    s = lax.axis_index("s")
    rows = (r0, r1, r2, r3)
    gsem = (g0, g1, g2, g3)
    ssem = (s0, s1, s2, s3)

    def zfill(r, carry):
        for k in range(8):
            zbuf_v[r, pl.ds(k * 16, 16)] = jnp.zeros((16,), jnp.float32)
        return carry

    lax.fori_loop(0, 32, zfill, 0)

    cbase = c * HALF
    pltpu.sync_copy(cnt_hbm.at[2 * s], cnt_v.at[pl.ds(0, NBKT * 128)])
    pltpu.sync_copy(cnt_hbm.at[2 * s + 1],
                    cnt_v.at[pl.ds(NBKT * 128, NBKT * 128)])

    def chunk_loop(f, carry0):
        pp = pp_hbm.at[f]
        dummy = pp.at[pl.ds(0, 64)]
        for z in range(10):
            pltpu.sync_copy(zbuf_v, acc_sh.at[pl.ds(s * 328 + z * 32, 32)])
        pltpu.sync_copy(zbuf_v.at[pl.ds(0, 8)],
                        acc_sh.at[pl.ds(s * 328 + 320, 8)])

        def quarter_loop(sg, carry1):
            g = 4 * c + sg
            pltpu.sync_copy(pp.at[pl.ds(sg * QTR + s * 160, 160)],
                            tbl_sh.at[pl.ds(s * 160, 160)])
            plsc.subcore_barrier()
            for j in range(2):
                w = 2 * s + j
                pltpu.sync_copy(psrc_hbm.at[w, g], sidx_v)
                pltpu.sync_copy(pdst_hbm.at[w, g], didx_v)
                n = cnt_v[pl.ds(j * NBKT * 128 + g * 128, 16)][0]
                nb = (n + 63) // 64

                def ring(o, carry2):
                    for q in range(NSLOT):
                        b = o * NSLOT + q

                        @pl.when(b < nb)
                        def _(b=b, q=q):
                            @pl.when(b >= NSLOT)
                            def _():
                                pltpu.make_async_copy(
                                    dummy, rows[q], ssem[q]).wait()
                            pltpu.async_copy(
                                tbl_sh.at[sidx_v.at[b]], rows[q], gsem[q])

                    for q in range(NSLOT):
                        b = o * NSLOT + q

                        @pl.when(b < nb)
                        def _(b=b, q=q):
                            pltpu.make_async_copy(
                                dummy, rows[q], gsem[q]).wait()
                            pltpu.async_copy(
                                rows[q], acc_sh.at[didx_v.at[b]],
                                ssem[q], add=True)

                    return carry2

                lax.fori_loop(0, (nb + NSLOT - 1) // NSLOT, ring, 0)
                for q in range(NSLOT):
                    @pl.when(q < nb)
                    def _(q=q):
                        pltpu.make_async_copy(dummy, rows[q],
                                              ssem[q]).wait()
            plsc.subcore_barrier()
            return carry1

        lax.fori_loop(0, 4, quarter_loop, 0)
        pltpu.sync_copy(
            acc_sh.at[pl.ds(s * 320, 320)],
            out_hbm.at[f].at[pl.ds(cbase + s * 320, 320)])
        plsc.subcore_barrier()
        return carry0

    lax.fori_loop(0, NCH, chunk_loop, 0)


@functools.cache
def _agg_call():
    return pl.kernel(
        _agg_body,
        out_type=jax.ShapeDtypeStruct((NCH, NP, 128), jnp.float32),
        mesh=_mesh(),
        scratch_types=[
            pltpu.VMEM((CAP // 64, 64), jnp.int32),
            pltpu.VMEM((CAP // 64, 64), jnp.int32),
            pltpu.VMEM((2 * NBKT * 128,), jnp.int32),
            pltpu.VMEM((64, 128), jnp.float32),
            pltpu.VMEM((64, 128), jnp.float32),
            pltpu.VMEM((64, 128), jnp.float32),
            pltpu.VMEM((64, 128), jnp.float32),
            pltpu.VMEM((32, 128), jnp.float32),
            pltpu.VMEM_SHARED((QTR, 128), jnp.float32),
            pltpu.VMEM_SHARED((ACCR, 128), jnp.float32),
            pltpu.SemaphoreType.DMA,
            pltpu.SemaphoreType.DMA,
            pltpu.SemaphoreType.DMA,
            pltpu.SemaphoreType.DMA,
            pltpu.SemaphoreType.DMA,
            pltpu.SemaphoreType.DMA,
            pltpu.SemaphoreType.DMA,
            pltpu.SemaphoreType.DMA,
        ],
    )


def _dinv_body(dp_ref, out_ref):
    out_ref[...] = lax.rsqrt(jnp.sum(dp_ref[...], axis=0) + 1.0)


def _dinv_call(dp):
    return pl.pallas_call(
        _dinv_body,
        out_shape=jax.ShapeDtypeStruct((NP // 128, 128), jnp.float32),
    )(dp)


def _w_body(sp_ref, dinv_ref, out_ref):
    ssum = jnp.sum(sp_ref[...], axis=0)
    dinv = dinv_ref[...]
    r = lax.broadcasted_iota(jnp.int32, (NP // 128, 128), 0)
    l = lax.broadcasted_iota(jnp.int32, (NP // 128, 128), 1)
    valid = (r * 128 + l) < N
    w = dinv * (ssum + dinv) * (1.0 / N)
    out_ref[...] = jnp.where(valid, w, 0.0)


def _w_call(sp, dinv):
    return pl.pallas_call(
        _w_body,
        out_shape=jax.ShapeDtypeStruct((NP // 128, 128), jnp.float32),
    )(sp, dinv)


def _mm_body(h_ref, wcat_ref, dvr_ref, lb_ref, pp_ref, r_ref):
    prod = jnp.dot(h_ref[...], wcat_ref[...], preferred_element_type=jnp.float32)
    dvr = dvr_ref[...]
    for f in range(NCH):
        pp_ref[f] = prod[:, 128 * f:128 * (f + 1)] * dvr
    r_ref[...] = prod[:, DH:] + lb_ref[...].reshape(1, DH)


def _mm_call(h, wcat, dvr, lb):
    k = h.shape[1]
    return pl.pallas_call(
        _mm_body,
        grid=(GR,),
        in_specs=[
            pl.BlockSpec((RB, k), lambda i: (i, 0)),
            pl.BlockSpec((k, DH * 2), lambda i: (0, 0)),
            pl.BlockSpec((RB, 128), lambda i: (i, 0)),
            pl.BlockSpec((1, 1, DH), lambda i: (0, 0, 0)),
        ],
        out_specs=(
            pl.BlockSpec((NCH, RB, 128), lambda i: (0, i, 0)),
            pl.BlockSpec((RB, DH), lambda i: (i, 0)),
        ),
        out_shape=(
            jax.ShapeDtypeStruct((NCH, NP, 128), jnp.float32),
            jax.ShapeDtypeStruct((NP, DH), jnp.float32),
        ),
        compiler_params=pltpu.CompilerParams(dimension_semantics=("arbitrary",)),
    )(h, wcat, dvr, lb)


def _bn_body(sc_ref, pp_ref, dvr_ref, r_ref, g_ref, be_ref, wrep_ref,
             h_ref, wsum_ref, stats, *, last):
    p = pl.program_id(0)
    i = pl.program_id(1)
    dvr = dvr_ref[...]
    y = jnp.concatenate(
        [(sc_ref[f] + pp_ref[f]) * dvr for f in range(NCH)], axis=1)

    @pl.when(jnp.logical_and(p == 0, i == 0))
    def _():
        stats[...] = jnp.zeros_like(stats)

    @pl.when(p == 0)
    def _():
        rows = i * RB + lax.broadcasted_iota(jnp.int32, (RB, DH), 0)
        ym = jnp.where(rows < N, y, 0.0)
        stats[0:1, :] = stats[0:1, :] + jnp.sum(ym, axis=0, keepdims=True)
        stats[1:2, :] = stats[1:2, :] + jnp.sum(ym * ym, axis=0, keepdims=True)

    @pl.when(p == 1)
    def _():
        m = stats[0:1, :] * (1.0 / N)
        v = stats[1:2, :] * (1.0 / N) - m * m
        rstd = lax.rsqrt(v + EPS)
        g = g_ref[...].reshape(1, DH)
        be = be_ref[...].reshape(1, DH)
        hn = jnp.maximum((y - m) * rstd * g + be, 0.0) + r_ref[...]
        h_ref[...] = hn
        if last:
            wr = wrep_ref[...]
            wcat = jnp.concatenate([wr, wr, wr, wr], axis=1)
            stats[2:3, :] = stats[2:3, :] + jnp.sum(hn * wcat, axis=0, keepdims=True)

            @pl.when(i == GR - 1)
            def _():
                wsum_ref[...] = stats[...]


def _bn_call(scs, pps, dvr, r, g, be, wrep, *, last):
    def body(sc_ref, pp_ref, dv, rr, gg, bb, *rest):
        if last:
            (wrp, h_ref, wsum_ref, stats) = rest
        else:
            (h_ref, stats) = rest
            wrp, wsum_ref = None, None
        _bn_body(sc_ref, pp_ref, dv, rr, gg, bb, wrp,
                 h_ref, wsum_ref, stats, last=last)

    chunk_spec = pl.BlockSpec((RB, 128), lambda p, i: (i, 0))
    stack_spec = pl.BlockSpec((NCH, RB, 128), lambda p, i: (0, i, 0))
    in_specs = (
        [stack_spec, stack_spec]
        + [chunk_spec,
           pl.BlockSpec((RB, DH), lambda p, i: (i, 0)),
           pl.BlockSpec((1, 1, DH), lambda p, i: (0, 0, 0)),
           pl.BlockSpec((1, 1, DH), lambda p, i: (0, 0, 0))]
    )
    out_specs = [pl.BlockSpec((RB, DH), lambda p, i: (p * i, 0))]
    out_shape = [jax.ShapeDtypeStruct((NP, DH), jnp.float32)]
    args = [scs, pps, dvr, r, g, be]
    if last:
        in_specs.append(chunk_spec)
        args.append(wrep)
        out_specs.append(pl.BlockSpec((8, DH), lambda p, i: (0, 0)))
        out_shape.append(jax.ShapeDtypeStruct((8, DH), jnp.float32))
    res = pl.pallas_call(
        body,
        grid=(2, GR),
        in_specs=in_specs,
        out_specs=tuple(out_specs),
        out_shape=tuple(out_shape),
        scratch_shapes=[pltpu.VMEM((8, DH), jnp.float32)],
        compiler_params=pltpu.CompilerParams(
            dimension_semantics=("arbitrary", "arbitrary")),
    )(*args)
    return res if last else (res[0], None)


def _fin_body(ws_ref, w3_ref, b3_ref, out_ref):
    v = ws_ref[2:3, :]
    res = jnp.dot(v, w3_ref[...], preferred_element_type=jnp.float32)
    res = res + b3_ref[...].reshape(1, DOUT)
    out_ref[...] = jnp.broadcast_to(res, (8, DOUT))


def _fin_call(wsum, W3, b3):
    return pl.pallas_call(
        _fin_body,
        out_shape=jax.ShapeDtypeStruct((8, DOUT), jnp.float32),
    )(wsum, W3, b3)


def kernel(x, edge_index, W0, b0, W1, b1, W2, b2, W3, b3,
           g0, be0, g1, be1, g2, be2, Lw0, Lb0, Lw1, Lb1, Lw2, Lb2):
    src = edge_index[0]
    dst = edge_index[1]
    src_w = src.reshape(NC, NS, EPW)
    dst_w = dst.reshape(NC, NS, EPW)
    idx_w = edge_index.reshape(2, NC, NS, EPW)
    psrc, pdst, cnts = _part_call()(idx_w)
    psrc = psrc.reshape(NW, NBKT, CAP // 64, 64)
    pdst = pdst.reshape(NW, NBKT, CAP // 64, 64)

    deg_parts = _deg_call()(dst_w)
    dinv2d = _dinv_call(deg_parts.reshape(NC * NS, NP // 128, 128))
    dinv_lin = dinv2d.reshape(NP)
    dvr = jnp.broadcast_to(dinv_lin[:, None], (NP, 128))
    s_parts = _s_call()(src_w, dst_w, dinv_lin)
    w2d = _w_call(s_parts.reshape(NC * NS, NP // 128, 128), dinv2d)
    wrep = jnp.broadcast_to(w2d.reshape(NP)[:, None], (NP, 128))

    h = jnp.pad(x, ((0, NP - N), (0, 0)))
    layer_params = ((W0, Lw0, Lb0, g0, be0),
                    (W1, Lw1, Lb1, g1, be1),
                    (W2, Lw2, Lb2, g2, be2))
    wsum = None
    for li, (W, Lw, Lb, g, be) in enumerate(layer_params):
        wcat = jnp.concatenate([W, Lw], axis=1)
        pps, r = _mm_call(h, wcat, dvr, Lb.reshape(1, 1, DH))
        scs = _agg_call()(pps, psrc, pdst, cnts)
        h, wsum = _bn_call(scs, pps, dvr, r, g.reshape(1, 1, DH),
                           be.reshape(1, 1, DH), wrep, last=(li == 2))

    out = _fin_call(wsum, W3, b3.reshape(1, 1, DOUT))
    return out[0:1]
